# baseline (speedup 1.0000x reference)
import functools

import jax
import jax.numpy as jnp
from jax.experimental import pallas as pl
from jax.experimental.pallas import tpu as pltpu

_P = 3
_H = 32
_EPS = 1e-5
_CORES = 2
_TN = 8192
_SPLIT = 2


def _emb_kernel(w1a_ref, w1b_ref, b1_ref, w2_ref, b2_ref,
                p1_ref, p2_ref, out_ref,
                x1_s, x2_s, o_s, in_sems, out_sems, *, nsteps):
    c = pl.program_id(0)
    s = pl.program_id(1)
    slot = jax.lax.rem(s, 2)
    half = _TN // _SPLIT

    def start_in(step, slot_t):
        r0 = (c * nsteps + step) * _TN
        for u in range(_SPLIT):
            pltpu.make_async_copy(
                p1_ref.at[pl.ds(r0 + u * half, half), :],
                x1_s.at[slot_t, pl.ds(u * half, half), :],
                in_sems.at[slot_t, 0, u]).start()
            pltpu.make_async_copy(
                p2_ref.at[pl.ds(r0 + u * half, half), :],
                x2_s.at[slot_t, pl.ds(u * half, half), :],
                in_sems.at[slot_t, 1, u]).start()

    @pl.when(s == 0)
    def _():
        start_in(s, slot)

    @pl.when(s + 1 < nsteps)
    def _():
        start_in(s + 1, 1 - slot)

    def wait_in(slot_t):
        for u in range(_SPLIT):
            pltpu.make_async_copy(
                p1_ref.at[pl.ds(0, half), :],
                x1_s.at[slot_t, pl.ds(0, half), :],
                in_sems.at[slot_t, 0, u]).wait()
            pltpu.make_async_copy(
                p2_ref.at[pl.ds(0, half), :],
                x2_s.at[slot_t, pl.ds(0, half), :],
                in_sems.at[slot_t, 1, u]).wait()

    wait_in(slot)

    def wait_out(slot_t):
        for u in range(_SPLIT):
            pltpu.make_async_copy(
                o_s.at[slot_t, pl.ds(0, half), :],
                out_ref.at[pl.ds(0, half), :],
                out_sems.at[slot_t, u]).wait()

    @pl.when(s >= 2)
    def _():
        wait_out(slot)

    h = jnp.dot(x1_s[slot], w1a_ref[...], preferred_element_type=jnp.float32)
    h += jnp.dot(x2_s[slot], w1b_ref[...], preferred_element_type=jnp.float32)
    h = jnp.maximum(h + b1_ref[...], 0.0)
    o_s[slot] = jnp.dot(h, w2_ref[...],
                        preferred_element_type=jnp.float32) + b2_ref[...]

    r0 = (c * nsteps + s) * _TN
    for u in range(_SPLIT):
        pltpu.make_async_copy(
            o_s.at[slot, pl.ds(u * half, half), :],
            out_ref.at[pl.ds(r0 + u * half, half), :],
            out_sems.at[slot, u]).start()

    @pl.when(s == nsteps - 1)
    def _():
        wait_out(1 - slot)
        wait_out(slot)


def _emb_fallback(pos1, pos2, w1a_eff, w1b_eff, b1_eff, w2_eff, b2_eff):
    n, p = pos1.shape
    tn = min(16384, n)

    def body(p1_ref, p2_ref, w1a_ref, w1b_ref, b1_ref, w2_ref, b2_ref, o_ref):
        h = jnp.dot(p1_ref[...], w1a_ref[...],
                    preferred_element_type=jnp.float32)
        h += jnp.dot(p2_ref[...], w1b_ref[...],
                     preferred_element_type=jnp.float32)
        h = jnp.maximum(h + b1_ref[...], 0.0)
        o_ref[...] = jnp.dot(h, w2_ref[...],
                             preferred_element_type=jnp.float32) + b2_ref[...]

    return pl.pallas_call(
        body,
        out_shape=jax.ShapeDtypeStruct((n, _H), jnp.float32),
        grid=(pl.cdiv(n, tn),),
        in_specs=[
            pl.BlockSpec((tn, p), lambda i: (i, 0)),
            pl.BlockSpec((tn, p), lambda i: (i, 0)),
            pl.BlockSpec((p, _H), lambda i: (0, 0)),
            pl.BlockSpec((p, _H), lambda i: (0, 0)),
            pl.BlockSpec((1, _H), lambda i: (0, 0)),
            pl.BlockSpec((_H, _H), lambda i: (0, 0)),
            pl.BlockSpec((1, _H), lambda i: (0, 0)),
        ],
        out_specs=pl.BlockSpec((tn, _H), lambda i: (i, 0)),
        compiler_params=pltpu.CompilerParams(
            dimension_semantics=("parallel",)),
    )(pos1, pos2, w1a_eff, w1b_eff, b1_eff, w2_eff, b2_eff)


@jax.jit
def kernel(pos1, pos2, w1, b1, w2, b2,
           bn1_gamma, bn1_beta, bn1_mean, bn1_var,
           bn2_gamma, bn2_beta, bn2_mean, bn2_var):
    n, p = pos1.shape

    s1 = bn1_gamma / jnp.sqrt(bn1_var + _EPS)
    t1 = bn1_beta - bn1_mean * s1
    s2 = bn2_gamma / jnp.sqrt(bn2_var + _EPS)
    t2 = bn2_beta - bn2_mean * s2

    w1a, w1b, w1c = w1[:, :p], w1[:, p:2 * p], w1[:, 2 * p:]
    w1a_eff = (w1a + w1c).T * s1[None, :]
    w1b_eff = (w1b - w1c).T * s1[None, :]
    b1_eff = (b1 * s1 + t1)[None, :]
    w2_eff = w2.T * s2[None, :]
    b2_eff = (b2 * s2 + t2)[None, :]

    if p != _P or w2.shape != (_H, _H) or n % (_CORES * _TN) != 0:
        return _emb_fallback(pos1, pos2, w1a_eff, w1b_eff,
                             b1_eff, w2_eff, b2_eff)

    nsteps = n // (_CORES * _TN)
    body = functools.partial(_emb_kernel, nsteps=nsteps)
    return pl.pallas_call(
        body,
        out_shape=jax.ShapeDtypeStruct((n, _H), jnp.float32),
        grid=(_CORES, nsteps),
        in_specs=[
            pl.BlockSpec((_P, _H), lambda c, s: (0, 0)),
            pl.BlockSpec((_P, _H), lambda c, s: (0, 0)),
            pl.BlockSpec((1, _H), lambda c, s: (0, 0)),
            pl.BlockSpec((_H, _H), lambda c, s: (0, 0)),
            pl.BlockSpec((1, _H), lambda c, s: (0, 0)),
            pl.BlockSpec(memory_space=pl.ANY),
            pl.BlockSpec(memory_space=pl.ANY),
        ],
        out_specs=pl.BlockSpec(memory_space=pl.ANY),
        scratch_shapes=[
            pltpu.VMEM((2, _TN, _P), jnp.float32),
            pltpu.VMEM((2, _TN, _P), jnp.float32),
            pltpu.VMEM((2, _TN, _H), jnp.float32),
            pltpu.SemaphoreType.DMA((2, 2, _SPLIT)),
            pltpu.SemaphoreType.DMA((2, _SPLIT)),
        ],
        compiler_params=pltpu.CompilerParams(
            dimension_semantics=("parallel", "arbitrary")),
    )(w1a_eff, w1b_eff, b1_eff, w2_eff, b2_eff, pos1, pos2)


# speedup vs baseline: 1.0235x; 1.0235x over previous
import jax
import jax.numpy as jnp
from jax.experimental import pallas as pl
from jax.experimental.pallas import tpu as pltpu

_P = 3
_H = 32
_EPS = 1e-5
_TN = 16384


def _emb_kernel(pos1_ref, pos2_ref, w1a_ref, w1b_ref, b1_ref, w2_ref, b2_ref,
                out_ref):
    h = jnp.dot(pos1_ref[...], w1a_ref[...],
                preferred_element_type=jnp.float32)
    h += jnp.dot(pos2_ref[...], w1b_ref[...],
                 preferred_element_type=jnp.float32)
    h = jnp.maximum(h + b1_ref[...], 0.0)
    out_ref[...] = jnp.dot(h, w2_ref[...],
                           preferred_element_type=jnp.float32) + b2_ref[...]


@jax.jit
def kernel(pos1, pos2, w1, b1, w2, b2,
           bn1_gamma, bn1_beta, bn1_mean, bn1_var,
           bn2_gamma, bn2_beta, bn2_mean, bn2_var):
    n, p = pos1.shape

    s1 = bn1_gamma / jnp.sqrt(bn1_var + _EPS)
    t1 = bn1_beta - bn1_mean * s1
    s2 = bn2_gamma / jnp.sqrt(bn2_var + _EPS)
    t2 = bn2_beta - bn2_mean * s2

    w1a, w1b, w1c = w1[:, :p], w1[:, p:2 * p], w1[:, 2 * p:]
    w1a_eff = (w1a + w1c).T * s1[None, :]
    w1b_eff = (w1b - w1c).T * s1[None, :]
    b1_eff = (b1 * s1 + t1)[None, :]
    w2_eff = w2.T * s2[None, :]
    b2_eff = (b2 * s2 + t2)[None, :]

    tn = min(_TN, n)
    grid = (pl.cdiv(n, tn),)
    return pl.pallas_call(
        _emb_kernel,
        out_shape=jax.ShapeDtypeStruct((n, _H), jnp.float32),
        grid=grid,
        in_specs=[
            pl.BlockSpec((tn, p), lambda i: (i, 0)),
            pl.BlockSpec((tn, p), lambda i: (i, 0)),
            pl.BlockSpec((p, _H), lambda i: (0, 0)),
            pl.BlockSpec((p, _H), lambda i: (0, 0)),
            pl.BlockSpec((1, _H), lambda i: (0, 0)),
            pl.BlockSpec((_H, _H), lambda i: (0, 0)),
            pl.BlockSpec((1, _H), lambda i: (0, 0)),
        ],
        out_specs=pl.BlockSpec((tn, _H), lambda i: (i, 0)),
        compiler_params=pltpu.CompilerParams(
            dimension_semantics=("parallel",)),
    )(pos1, pos2, w1a_eff, w1b_eff, b1_eff, w2_eff, b2_eff)
